# baseline (speedup 1.0000x reference)
import functools

import jax
import jax.numpy as jnp
from jax import lax
from jax.experimental import pallas as pl
from jax.experimental.pallas import tpu as pltpu
from jax.experimental.pallas import tpu_sc as plsc

_L = 16
_NS = 16
_NC = 1


def _permute(x, perm, dn):
  return lax.gather(x, perm[:, None], dn, slice_sizes=(1,),
                    mode=lax.GatherScatterMode.PROMISE_IN_BOUNDS)


def _sc_body(B, S, H, enc_hbm, rowids_hbm, out_hbm, idx_v, rows_v, out_v,
             sem):
  sid = lax.axis_index("s")
  nq = _NS // B
  b = sid // nq
  q = sid % nq

  pltpu.sync_copy(rowids_hbm, idx_v)
  pltpu.async_copy(enc_hbm.at[idx_v.at[b]], rows_v, sem).wait()

  lanes = lax.iota(jnp.int32, _L)
  lo = lanes < (_L // 2)
  pe = (2 * lanes) % _L
  po = (2 * lanes + 1) % _L
  dn = lax.GatherDimensionNumbers(
      offset_dims=(), collapsed_slice_dims=(0,), start_index_map=(0,))

  vper = H // (2 * _L * nq)
  dbase = pl.multiple_of(2 * _L * vper * q, 2 * _L)
  obase = pl.multiple_of(_L * vper * q, _L)
  for v in range(vper):
    yi0 = rows_v[0, pl.ds(dbase + 2 * _L * v, _L)]
    yj0 = rows_v[1, pl.ds(dbase + 2 * _L * v, _L)]
    yi1 = rows_v[0, pl.ds(dbase + 2 * _L * v + _L, _L)]
    yj1 = rows_v[1, pl.ds(dbase + 2 * _L * v + _L, _L)]
    d0 = yj0 - yi0
    d1 = yj1 - yi1
    even = jnp.where(lo, _permute(d0, pe, dn), _permute(d1, pe, dn))
    odd = jnp.where(lo, _permute(d0, po, dn), _permute(d1, po, dn))
    out_v[pl.ds(_L * v, _L)] = even
    out_v[pl.ds(_L * vper + _L * v, _L)] = -odd

  width = _L * vper
  off_e = pl.multiple_of(b * H + obase, _L)
  off_o = pl.multiple_of(b * H + H // 2 + obase, _L)
  pltpu.sync_copy(out_v.at[pl.ds(0, width)], out_hbm.at[pl.ds(off_e, width)])
  pltpu.sync_copy(out_v.at[pl.ds(width, width)],
                  out_hbm.at[pl.ds(off_o, width)])


@functools.partial(jax.jit, static_argnums=(2, 3, 4))
def _run(enc2d, rowids, B, S, H):
  mesh = plsc.VectorSubcoreMesh(core_axis_name="c", subcore_axis_name="s",
                                num_cores=_NC, num_subcores=_NS)
  body = functools.partial(_sc_body, B, S, H)
  nq = _NS // B
  fn = pl.kernel(
      body,
      out_type=jax.ShapeDtypeStruct((B * H,), jnp.float32),
      mesh=mesh,
      scratch_types=[
          pltpu.VMEM((2 * B, 2), jnp.int32),
          pltpu.VMEM((2, H), jnp.float32),
          pltpu.VMEM((H // nq,), jnp.float32),
          pltpu.SemaphoreType.DMA,
      ],
  )
  return fn(enc2d, rowids)


def kernel(encoded, pos):
  B, S, H = encoded.shape
  enc2d = encoded.reshape(B * S, H)
  base = jnp.arange(B, dtype=jnp.int32) * S
  rows = jnp.stack([base + pos[:, 0], base + pos[:, 1] - 1], axis=1)
  rowids = jnp.pad(rows.astype(jnp.int32), ((0, B), (0, 0)))
  out = _run(enc2d, rowids, B, S, H)
  return out.reshape(B, H)


# speedup vs baseline: 1.0082x; 1.0082x over previous
import functools

import jax
import jax.numpy as jnp
from jax import lax
from jax.experimental import pallas as pl
from jax.experimental.pallas import tpu as pltpu
from jax.experimental.pallas import tpu_sc as plsc

_L = 16
_NS = 16
_NC = 1


def _permute(x, perm, dn):
  return lax.gather(x, perm[:, None], dn, slice_sizes=(1,),
                    mode=lax.GatherScatterMode.PROMISE_IN_BOUNDS)


def _sc_body(B, S, H, enc_hbm, rowids_hbm, out_hbm, idx_v, rows_v, out_v,
             sem):
  sid = lax.axis_index("s")
  nq = _NS // B
  b = sid // nq
  r = sid % nq
  p = r // 2
  seg = r % 2

  pltpu.sync_copy(rowids_hbm, idx_v)
  pltpu.async_copy(enc_hbm.at[idx_v.at[b]], rows_v, sem).wait()

  lanes = lax.iota(jnp.int32, _L)
  lo = lanes < (_L // 2)
  pp = (2 * lanes + p) % _L
  sgn = (1 - 2 * p).astype(jnp.float32)
  dn = lax.GatherDimensionNumbers(
      offset_dims=(), collapsed_slice_dims=(0,), start_index_map=(0,))

  vper = H // (4 * _L)
  dbase = pl.multiple_of((H // 2) * seg, 2 * _L)
  for v in range(vper):
    yi0 = rows_v[0, pl.ds(dbase + 2 * _L * v, _L)]
    yj0 = rows_v[1, pl.ds(dbase + 2 * _L * v, _L)]
    yi1 = rows_v[0, pl.ds(dbase + 2 * _L * v + _L, _L)]
    yj1 = rows_v[1, pl.ds(dbase + 2 * _L * v + _L, _L)]
    d0 = yj0 - yi0
    d1 = yj1 - yi1
    out_v[pl.ds(_L * v, _L)] = sgn * jnp.where(
        lo, _permute(d0, pp, dn), _permute(d1, pp, dn))

  width = _L * vper
  off = pl.multiple_of(b * H + p * (H // 2) + width * seg, width)
  pltpu.sync_copy(out_v, out_hbm.at[pl.ds(off, width)])


@functools.partial(jax.jit, static_argnums=(2, 3, 4))
def _run(enc2d, rowids, B, S, H):
  mesh = plsc.VectorSubcoreMesh(core_axis_name="c", subcore_axis_name="s",
                                num_cores=_NC, num_subcores=_NS)
  body = functools.partial(_sc_body, B, S, H)
  nq = _NS // B
  fn = pl.kernel(
      body,
      out_type=jax.ShapeDtypeStruct((B * H,), jnp.float32),
      mesh=mesh,
      scratch_types=[
          pltpu.VMEM((2 * B, 2), jnp.int32),
          pltpu.VMEM((2, H), jnp.float32),
          pltpu.VMEM((H // 4,), jnp.float32),
          pltpu.SemaphoreType.DMA,
      ],
  )
  return fn(enc2d, rowids)


def kernel(encoded, pos):
  B, S, H = encoded.shape
  enc2d = encoded.reshape(B * S, H)
  base = jnp.arange(B, dtype=jnp.int32) * S
  rows = jnp.stack([base + pos[:, 0], base + pos[:, 1] - 1], axis=1)
  rowids = jnp.pad(rows.astype(jnp.int32), ((0, B), (0, 0)))
  out = _run(enc2d, rowids, B, S, H)
  return out.reshape(B, H)


# speedup vs baseline: 1.0504x; 1.0418x over previous
import functools

import jax
import jax.numpy as jnp
from jax import lax
from jax.experimental import pallas as pl
from jax.experimental.pallas import tpu as pltpu
from jax.experimental.pallas import tpu_sc as plsc

_L = 16
_NS = 16
_NC = 1


def _permute(x, perm, dn):
  return lax.gather(x, perm[:, None], dn, slice_sizes=(1,),
                    mode=lax.GatherScatterMode.PROMISE_IN_BOUNDS)


def _sc_body(B, S, H, enc_hbm, rowids_hbm, out_hbm, idx_v, rows_v, out_v,
             sem):
  sid = lax.axis_index("s")
  nq = _NS // B
  b = sid // nq
  r = sid % nq
  p = r // 2
  seg = r % 2

  pltpu.sync_copy(rowids_hbm, idx_v)
  pltpu.async_copy(enc_hbm.at[idx_v.at[b]], rows_v, sem).wait()

  lanes = lax.iota(jnp.int32, _L)
  lo = lanes < (_L // 2)
  pp = (2 * lanes + p) % _L
  sgn = (1 - 2 * p).astype(jnp.float32)
  dn = lax.GatherDimensionNumbers(
      offset_dims=(), collapsed_slice_dims=(0,), start_index_map=(0,))

  vper = H // (4 * _L)
  dbase = pl.multiple_of((H // 2) * seg, 2 * _L)
  for v in range(vper):
    yi0 = rows_v[0, pl.ds(dbase + 2 * _L * v, _L)]
    yj0 = rows_v[1, pl.ds(dbase + 2 * _L * v, _L)]
    yi1 = rows_v[0, pl.ds(dbase + 2 * _L * v + _L, _L)]
    yj1 = rows_v[1, pl.ds(dbase + 2 * _L * v + _L, _L)]
    d0 = yj0 - yi0
    d1 = yj1 - yi1
    out_v[pl.ds(_L * v, _L)] = sgn * jnp.where(
        lo, _permute(d0, pp, dn), _permute(d1, pp, dn))

  width = _L * vper
  col = pl.multiple_of(p * (H // 2) + width * seg, width)
  pltpu.sync_copy(out_v, out_hbm.at[b].at[pl.ds(col, width)])


@functools.partial(jax.jit, static_argnums=(2, 3, 4))
def _run(enc2d, rowids, B, S, H):
  mesh = plsc.VectorSubcoreMesh(core_axis_name="c", subcore_axis_name="s",
                                num_cores=_NC, num_subcores=_NS)
  body = functools.partial(_sc_body, B, S, H)
  fn = pl.kernel(
      body,
      out_type=jax.ShapeDtypeStruct((B, H), jnp.float32),
      mesh=mesh,
      scratch_types=[
          pltpu.VMEM((2 * B, 2), jnp.int32),
          pltpu.VMEM((2, H), jnp.float32),
          pltpu.VMEM((H // 4,), jnp.float32),
          pltpu.SemaphoreType.DMA,
      ],
  )
  return fn(enc2d, rowids)


def kernel(encoded, pos):
  B, S, H = encoded.shape
  enc2d = encoded.reshape(B * S, H)
  base = jnp.arange(B, dtype=jnp.int32)[:, None] * S
  base = base - jnp.array([[0, 1]], jnp.int32)
  rowids = jnp.pad(pos.astype(jnp.int32) + base, ((0, B), (0, 0)))
  return _run(enc2d, rowids, B, S, H)


# speedup vs baseline: 1.0866x; 1.0345x over previous
import functools

import jax
import jax.numpy as jnp
from jax import lax
from jax.experimental import pallas as pl
from jax.experimental.pallas import tpu as pltpu
from jax.experimental.pallas import tpu_sc as plsc

_L = 16
_NS = 16
_NC = 1


def _permute(x, perm, dn):
  return lax.gather(x, perm[:, None], dn, slice_sizes=(1,),
                    mode=lax.GatherScatterMode.PROMISE_IN_BOUNDS)


def _sc_body(B, S, H, enc_hbm, rowids_hbm, out_hbm, idx_v, rows_v, out_v,
             sem):
  sid = lax.axis_index("s")
  nq = _NS // B
  b = sid // nq
  r = sid % nq
  p = r // 2
  seg = r % 2

  pltpu.sync_copy(rowids_hbm, idx_v)
  pltpu.async_copy(enc_hbm.at[idx_v.at[b]], rows_v, sem).wait()

  lanes = lax.iota(jnp.int32, _L)
  lo = lanes < (_L // 2)
  pp = (2 * lanes + p) % _L
  sgn = (1 - 2 * p).astype(jnp.float32)
  dn = lax.GatherDimensionNumbers(
      offset_dims=(), collapsed_slice_dims=(0,), start_index_map=(0,))

  vper = H // (4 * _L)
  dbase = pl.multiple_of((H // 2) * seg, 2 * _L)

  def _step(v, carry):
    yi0 = rows_v[0, pl.ds(dbase + 2 * _L * v, _L)]
    yj0 = rows_v[1, pl.ds(dbase + 2 * _L * v, _L)]
    yi1 = rows_v[0, pl.ds(dbase + 2 * _L * v + _L, _L)]
    yj1 = rows_v[1, pl.ds(dbase + 2 * _L * v + _L, _L)]
    d0 = yj0 - yi0
    d1 = yj1 - yi1
    out_v[pl.ds(_L * v, _L)] = sgn * jnp.where(
        lo, _permute(d0, pp, dn), _permute(d1, pp, dn))
    return carry

  lax.fori_loop(0, vper, _step, 0)

  width = _L * vper
  col = pl.multiple_of(p * (H // 2) + width * seg, width)
  pltpu.sync_copy(out_v, out_hbm.at[b].at[pl.ds(col, width)])


@functools.partial(jax.jit, static_argnums=(2, 3, 4))
def _run(enc2d, rowids, B, S, H):
  mesh = plsc.VectorSubcoreMesh(core_axis_name="c", subcore_axis_name="s",
                                num_cores=_NC, num_subcores=_NS)
  body = functools.partial(_sc_body, B, S, H)
  fn = pl.kernel(
      body,
      out_type=jax.ShapeDtypeStruct((B, H), jnp.float32),
      mesh=mesh,
      scratch_types=[
          pltpu.VMEM((2 * B, 2), jnp.int32),
          pltpu.VMEM((2, H), jnp.float32),
          pltpu.VMEM((H // 4,), jnp.float32),
          pltpu.SemaphoreType.DMA,
      ],
  )
  return fn(enc2d, rowids)


def kernel(encoded, pos):
  B, S, H = encoded.shape
  enc2d = encoded.reshape(B * S, H)
  base = jnp.arange(B, dtype=jnp.int32)[:, None] * S
  base = base - jnp.array([[0, 1]], jnp.int32)
  rowids = jnp.pad(pos.astype(jnp.int32) + base, ((0, B), (0, 0)))
  return _run(enc2d, rowids, B, S, H)


# speedup vs baseline: 1.0996x; 1.0120x over previous
import functools

import jax
import jax.numpy as jnp
from jax import lax
from jax.experimental import pallas as pl
from jax.experimental.pallas import tpu as pltpu
from jax.experimental.pallas import tpu_sc as plsc

_L = 16
_NS = 16
_NC = 1


def _permute(x, perm, dn):
  return lax.gather(x, perm[:, None], dn, slice_sizes=(1,),
                    mode=lax.GatherScatterMode.PROMISE_IN_BOUNDS)


def _sc_body(B, S, H, enc_hbm, rowids_hbm, out_hbm, idx_v, rows_v, out_v,
             sem):
  sid = lax.axis_index("s")
  nq = _NS // B
  b = sid // nq
  r = sid % nq
  p = r // 2
  seg = r % 2

  pltpu.sync_copy(rowids_hbm, idx_v)
  pltpu.async_copy(enc_hbm.at[idx_v.at[b]], rows_v, sem).wait()

  lanes = lax.iota(jnp.int32, _L)
  lo = lanes < (_L // 2)
  pp = (2 * lanes + p) % _L
  sgn = (1 - 2 * p).astype(jnp.float32)
  dn = lax.GatherDimensionNumbers(
      offset_dims=(), collapsed_slice_dims=(0,), start_index_map=(0,))

  vper = H // (4 * _L)
  dbase = pl.multiple_of((H // 2) * seg, 2 * _L)

  def _step(v, carry):
    yi0 = rows_v[0, pl.ds(dbase + 2 * _L * v, _L)]
    yj0 = rows_v[1, pl.ds(dbase + 2 * _L * v, _L)]
    yi1 = rows_v[0, pl.ds(dbase + 2 * _L * v + _L, _L)]
    yj1 = rows_v[1, pl.ds(dbase + 2 * _L * v + _L, _L)]
    d0 = yj0 - yi0
    d1 = yj1 - yi1
    out_v[pl.ds(_L * v, _L)] = sgn * jnp.where(
        lo, _permute(d0, pp, dn), _permute(d1, pp, dn))
    return carry

  lax.fori_loop(0, vper, _step, 0)

  width = _L * vper
  col = pl.multiple_of(p * (H // 2) + width * seg, width)
  pltpu.sync_copy(out_v, out_hbm.at[b].at[pl.ds(col, width)])


@functools.partial(jax.jit, static_argnums=(2, 3, 4))
def _run(enc2d, rowids, B, S, H):
  mesh = plsc.VectorSubcoreMesh(core_axis_name="c", subcore_axis_name="s",
                                num_cores=_NC, num_subcores=_NS)
  body = functools.partial(_sc_body, B, S, H)
  fn = pl.kernel(
      body,
      out_type=jax.ShapeDtypeStruct((B, H), jnp.float32),
      mesh=mesh,
      scratch_types=[
          pltpu.VMEM((B, 2), jnp.int32),
          pltpu.VMEM((2, H), jnp.float32),
          pltpu.VMEM((H // 4,), jnp.float32),
          pltpu.SemaphoreType.DMA,
      ],
  )
  return fn(enc2d, rowids)


def kernel(encoded, pos):
  B, S, H = encoded.shape
  enc2d = encoded.reshape(B * S, H)
  base = jnp.arange(B, dtype=jnp.int32)[:, None] * S
  base = base - jnp.array([[0, 1]], jnp.int32)
  rowids = pos.astype(jnp.int32) + base
  return _run(enc2d, rowids, B, S, H)


# speedup vs baseline: 1.1000x; 1.0004x over previous
import functools

import jax
import jax.numpy as jnp
import numpy as np
from jax import lax
from jax.experimental import pallas as pl
from jax.experimental.pallas import tpu as pltpu
from jax.experimental.pallas import tpu_sc as plsc

_L = 16
_NS = 16
_NC = 1


def _permute(x, perm, dn):
  return lax.gather(x, perm[:, None], dn, slice_sizes=(1,),
                    mode=lax.GatherScatterMode.PROMISE_IN_BOUNDS)


def _sc_body(B, S, H, enc_hbm, rowids_hbm, out_hbm, idx_v, rows_v, out_v,
             sem):
  sid = lax.axis_index("s")
  nq = _NS // B
  b = sid // nq
  r = sid % nq
  p = r // 2
  seg = r % 2

  pltpu.sync_copy(rowids_hbm, idx_v)
  pltpu.async_copy(enc_hbm.at[idx_v.at[b]], rows_v, sem).wait()

  lanes = lax.iota(jnp.int32, _L)
  lo = lanes < (_L // 2)
  pp = (2 * lanes + p) % _L
  sgn = (1 - 2 * p).astype(jnp.float32)
  dn = lax.GatherDimensionNumbers(
      offset_dims=(), collapsed_slice_dims=(0,), start_index_map=(0,))

  vper = H // (4 * _L)
  dbase = pl.multiple_of((H // 2) * seg, 2 * _L)

  def _step(v, carry):
    yi0 = rows_v[0, pl.ds(dbase + 2 * _L * v, _L)]
    yj0 = rows_v[1, pl.ds(dbase + 2 * _L * v, _L)]
    yi1 = rows_v[0, pl.ds(dbase + 2 * _L * v + _L, _L)]
    yj1 = rows_v[1, pl.ds(dbase + 2 * _L * v + _L, _L)]
    d0 = yj0 - yi0
    d1 = yj1 - yi1
    out_v[pl.ds(_L * v, _L)] = sgn * jnp.where(
        lo, _permute(d0, pp, dn), _permute(d1, pp, dn))
    return carry

  lax.fori_loop(0, vper, _step, 0)

  width = _L * vper
  col = pl.multiple_of(p * (H // 2) + width * seg, width)
  pltpu.sync_copy(out_v, out_hbm.at[b].at[pl.ds(col, width)])


@functools.partial(jax.jit, static_argnums=(2, 3, 4))
def _run(enc2d, rowids, B, S, H):
  mesh = plsc.VectorSubcoreMesh(core_axis_name="c", subcore_axis_name="s",
                                num_cores=_NC, num_subcores=_NS)
  body = functools.partial(_sc_body, B, S, H)
  fn = pl.kernel(
      body,
      out_type=jax.ShapeDtypeStruct((B, H), jnp.float32),
      mesh=mesh,
      scratch_types=[
          pltpu.VMEM((B, 2), jnp.int32),
          pltpu.VMEM((2, H), jnp.float32),
          pltpu.VMEM((H // 4,), jnp.float32),
          pltpu.SemaphoreType.DMA,
      ],
  )
  return fn(enc2d, rowids)


def kernel(encoded, pos):
  B, S, H = encoded.shape
  enc2d = encoded.reshape(B * S, H)
  base = np.arange(B, dtype=np.int32)[:, None] * S - np.array([[0, 1]], np.int32)
  rowids = pos.astype(jnp.int32) + jnp.asarray(base)
  return _run(enc2d, rowids, B, S, H)
